# baseline (speedup 1.0000x reference)
import jax
import jax.numpy as jnp
from jax import lax
from jax.experimental import pallas as pl
from jax.experimental.pallas import tpu as pltpu
from jax.experimental.pallas import tpu_sc as plsc

_E, _K = 8, 2
_HBLK = 512
_TBLK = 1024
_WTOK = 128


def _gate_body(x_ref, wg_ref, bg_ref, lt_ref):
    logits = jax.lax.dot_general(
        wg_ref[...], x_ref[...], (((1,), (1,)), ((), ())),
        preferred_element_type=jnp.float32)
    lt_ref[...] = logits + bg_ref[...]


def _route_body(lt_hbm, s0_hbm, s1_hbm, p1_hbm, p2_hbm, lbuf, s0buf, s1buf,
                p1buf, p2buf):
    wid = lax.axis_index("s") * 2 + lax.axis_index("c")
    base = wid * _WTOK
    pltpu.sync_copy(lt_hbm.at[:, pl.ds(base, _WTOK)], lbuf)
    for j in range(_WTOK // 16):
        sl = pl.ds(j * 16, 16)
        v1 = lbuf[0, sl]
        i1 = jnp.zeros((16,), jnp.int32)
        v2 = jnp.full((16,), -1e30, jnp.float32)
        i2 = jnp.zeros((16,), jnp.int32)
        for e in range(1, _E):
            v = lbuf[e, sl]
            ev = jnp.full((16,), e, jnp.int32)
            v2n = jnp.where(v > v2, v, v2)
            i2n = jnp.where(v > v2, ev, i2)
            v2 = jnp.where(v > v1, v1, v2n)
            i2 = jnp.where(v > v1, i1, i2n)
            i1 = jnp.where(v > v1, ev, i1)
            v1 = jnp.where(v > v1, v, v1)
        p = jnp.exp(v2 - v1)
        d = 1.0 + p
        s0buf[sl] = 1.0 / d
        s1buf[sl] = p / d
        if j == 0:
            p1buf[...] = i1
            p2buf[...] = i2
    pltpu.sync_copy(s0buf, s0_hbm.at[pl.ds(base, _WTOK)])
    pltpu.sync_copy(s1buf, s1_hbm.at[pl.ds(base, _WTOK)])

    @pl.when(wid == 0)
    def _():
        pltpu.sync_copy(p1buf, p1_hbm)
        pltpu.sync_copy(p2buf, p2_hbm)


def _gelu(v):
    return v * 0.5 * (1.0 + jax.lax.erf(v * 0.7071067811865476))


def _moe_body(p1_ref, p2_ref, x_ref, w1a_ref, b1a_ref, w2a_ref, b2a_ref,
              w1b_ref, b1b_ref, w2b_ref, b2b_ref, g0_ref, g1_ref, o_ref):
    del p1_ref, p2_ref
    h = pl.program_id(2)
    xb = x_ref[...]
    g0 = g0_ref[...]
    g1 = g1_ref[...]

    def expert(w1_ref, b1_ref, w2_ref, g):
        hpre = jax.lax.dot_general(
            xb, w1_ref[0], (((1,), (1,)), ((), ())),
            preferred_element_type=jnp.float32)
        hact = _gelu(hpre + b1_ref[0]) * g
        return jax.lax.dot_general(
            hact, w2_ref[0], (((1,), (1,)), ((), ())),
            preferred_element_type=jnp.float32)

    contrib = (expert(w1a_ref, b1a_ref, w2a_ref, g0)
               + expert(w1b_ref, b1b_ref, w2b_ref, g1))

    @pl.when(h == 0)
    def _():
        o_ref[...] = contrib + g0 * b2a_ref[0] + g1 * b2b_ref[0]

    @pl.when(h != 0)
    def _():
        o_ref[...] = o_ref[...] + contrib


def kernel(x, W1, b1, W2, b2, Wg, bg):
    B, S, D = x.shape
    E, H, _ = W1.shape
    T = B * S
    x2 = x.reshape(T, D)
    bgc = bg.reshape(E, 1)

    logitsT = pl.pallas_call(
        _gate_body,
        grid=(T // _TBLK,),
        in_specs=[
            pl.BlockSpec((_TBLK, D), lambda g: (g, 0)),
            pl.BlockSpec((E, D), lambda g: (0, 0)),
            pl.BlockSpec((E, 1), lambda g: (0, 0)),
        ],
        out_specs=pl.BlockSpec((E, _TBLK), lambda g: (0, g)),
        out_shape=jax.ShapeDtypeStruct((E, T), jnp.float32),
    )(x2, Wg, bgc)

    mesh = plsc.VectorSubcoreMesh(core_axis_name="c", subcore_axis_name="s")
    s0f, s1f, p1, p2 = pl.kernel(
        _route_body,
        out_type=[
            jax.ShapeDtypeStruct((T,), jnp.float32),
            jax.ShapeDtypeStruct((T,), jnp.float32),
            jax.ShapeDtypeStruct((16,), jnp.int32),
            jax.ShapeDtypeStruct((16,), jnp.int32),
        ],
        scratch_types=[
            pltpu.VMEM((E, _WTOK), jnp.float32),
            pltpu.VMEM((_WTOK,), jnp.float32),
            pltpu.VMEM((_WTOK,), jnp.float32),
            pltpu.VMEM((16,), jnp.int32),
            pltpu.VMEM((16,), jnp.int32),
        ],
        mesh=mesh,
    )(logitsT)
    s0 = s0f.reshape(T, 1)
    s1 = s1f.reshape(T, 1)

    b1r = b1.reshape(E, 1, H)
    b2r = b2.reshape(E, 1, D)
    NH = H // _HBLK
    NS = 2
    SB = S // NS
    grid_spec = pltpu.PrefetchScalarGridSpec(
        num_scalar_prefetch=2,
        grid=(B, NS, NH),
        in_specs=[
            pl.BlockSpec((SB, D), lambda b, s, h, p1, p2: (b * 2 + s, 0)),
            pl.BlockSpec((1, _HBLK, D), lambda b, s, h, p1, p2: (p1[b], h, 0)),
            pl.BlockSpec((1, 1, _HBLK), lambda b, s, h, p1, p2: (p1[b], 0, h)),
            pl.BlockSpec((1, D, _HBLK), lambda b, s, h, p1, p2: (p1[b], 0, h)),
            pl.BlockSpec((1, 1, D), lambda b, s, h, p1, p2: (p1[b], 0, 0)),
            pl.BlockSpec((1, _HBLK, D), lambda b, s, h, p1, p2: (p2[b], h, 0)),
            pl.BlockSpec((1, 1, _HBLK), lambda b, s, h, p1, p2: (p2[b], 0, h)),
            pl.BlockSpec((1, D, _HBLK), lambda b, s, h, p1, p2: (p2[b], 0, h)),
            pl.BlockSpec((1, 1, D), lambda b, s, h, p1, p2: (p2[b], 0, 0)),
            pl.BlockSpec((SB, 1), lambda b, s, h, p1, p2: (b * 2 + s, 0)),
            pl.BlockSpec((SB, 1), lambda b, s, h, p1, p2: (b * 2 + s, 0)),
        ],
        out_specs=pl.BlockSpec((SB, D), lambda b, s, h, p1, p2: (b * 2 + s, 0)),
    )
    out = pl.pallas_call(
        _moe_body,
        grid_spec=grid_spec,
        out_shape=jax.ShapeDtypeStruct((T, D), jnp.float32),
        compiler_params=pltpu.CompilerParams(
            dimension_semantics=("parallel", "parallel", "arbitrary")),
    )(p1, p2, x2, W1, b1r, W2, b2r, W1, b1r, W2, b2r, s0, s1)
    return out.reshape(B, S, D)
